# baseline (speedup 1.0000x reference)
import functools

import jax
import jax.numpy as jnp
from jax.experimental import pallas as pl

B = 4096
D = 128
N = 100000
K = 50

NBLK = 1024
NCHUNK = 128
CP = 98 * NBLK
QBLK = 256
NEG = -3.0e38


def _mm_body(q_ref, c_ref, s_ref, m_ref):
    n = pl.program_id(0)
    s = jax.lax.dot_general(
        q_ref[...], c_ref[...], (((1,), (1,)), ((), ())),
        preferred_element_type=jnp.float32)
    col = n * NBLK + jax.lax.broadcasted_iota(jnp.int32, (QBLK, NBLK), 1)
    s = jnp.where(col < N, s, NEG)
    s_ref[...] = s
    m_ref[...] = jnp.max(s.reshape(QBLK, NBLK // NCHUNK, NCHUNK), axis=-1
                         ).reshape(1, QBLK, NBLK // NCHUNK)


def _matmul_scores(q, corpus_p):
    grid = (CP // NBLK, B // QBLK)
    return pl.pallas_call(
        _mm_body,
        grid=grid,
        in_specs=[
            pl.BlockSpec((QBLK, D), lambda n, qt: (qt, 0)),
            pl.BlockSpec((NBLK, D), lambda n, qt: (n, 0)),
        ],
        out_specs=[
            pl.BlockSpec((QBLK, NBLK), lambda n, qt: (qt, n)),
            pl.BlockSpec((1, QBLK, NBLK // NCHUNK), lambda n, qt: (n, qt, 0)),
        ],
        out_shape=[
            jax.ShapeDtypeStruct((B, CP), jnp.float32),
            jax.ShapeDtypeStruct((CP // NBLK, B, NBLK // NCHUNK), jnp.float32),
        ],
    )(q, corpus_p)


def kernel(query_embedding, num_items, corpus):
    corpus_p = jnp.pad(corpus, ((0, CP - N), (0, 0)))
    scores, _chunkmax = _matmul_scores(query_embedding, corpus_p)
    _, idx = jax.lax.top_k(scores, K)
    gathered = jnp.take(corpus, idx, axis=0)
    return idx, gathered
